# baseline (speedup 1.0000x reference)
import jax
import jax.numpy as jnp
from jax import lax
from jax.experimental import pallas as pl
from jax.experimental.pallas import tpu as pltpu
from jax.experimental.pallas import tpu_sc as plsc

B = 32
CLS = 100
D = 512
CTX_LEN = 12
POOL_G = 10
POOL_A = 100
SEQ = 77
N_CTX = 36
SUF = 40
NC_SUF = 64

NCORE = 2
NSUB = 16
NW = NCORE * NSUB
G_ROWS = B * CTX_LEN
GB = G_ROWS // N_CTX
GSP = G_ROWS - GB * N_CTX
CTX_PAD = 40

CLS_BLK = 50
NCB = CLS // CLS_BLK


def _sc_gather(g2d_hbm, a2d_hbm, idxg_hbm, idxa_hbm, ctx_hbm,
               idxg_v, idxa_v, gat_v, sem):
    cid = lax.axis_index("c")
    sid = lax.axis_index("s")
    w = sid * NCORE + cid

    pltpu.sync_copy(idxg_hbm.at[w, 0], idxg_v)
    pltpu.sync_copy(idxa_hbm.at[w, 0], idxa_v)

    @pl.when(w < GB)
    def _all_global():
        pltpu.async_copy(g2d_hbm.at[idxg_v], gat_v, sem).wait()

    @pl.when(w == GB)
    def _straddle():
        pltpu.async_copy(a2d_hbm.at[idxa_v.at[pl.ds(16, 24)]],
                         gat_v.at[pl.ds(16, 24)], sem).wait()
        pltpu.async_copy(g2d_hbm.at[idxg_v.at[pl.ds(0, GSP)]],
                         gat_v.at[pl.ds(0, GSP)], sem).wait()

    @pl.when(w > GB)
    def _all_attr():
        pltpu.async_copy(a2d_hbm.at[idxa_v], gat_v, sem).wait()

    pltpu.sync_copy(gat_v, ctx_hbm.at[w])


def _prompt_kernel(ctx_ref, pref_ref, suf_ref, tok_ref,
                   out_p_ref, out_t_ref):
    cb = pl.program_id(0)
    b = pl.program_id(1)
    c0 = cb * CLS_BLK

    ctx = ctx_ref[b][:N_CTX]
    full = jnp.concatenate([
        pref_ref[pl.ds(c0, CLS_BLK)],
        jnp.broadcast_to(ctx[None], (CLS_BLK, N_CTX, D)),
        suf_ref[pl.ds(c0, CLS_BLK)],
    ], axis=1)
    out_p_ref[...] = full
    out_t_ref[...] = tok_ref[pl.ds(c0, CLS_BLK)]


def _nc_kernel(g_ref, ncp_ref, ncs_ref, nct_ref, out_ncp_ref, out_nct_ref):
    out_ncp_ref[...] = jnp.concatenate([
        jnp.broadcast_to(ncp_ref[...], (POOL_G, 1, D)),
        g_ref[...],
        jnp.broadcast_to(ncs_ref[...], (POOL_G, NC_SUF, D)),
    ], axis=1)
    out_nct_ref[...] = jnp.broadcast_to(nct_ref[...], (POOL_G, SEQ))


@jax.jit
def _run(idx_g, idx_a, global_prompt, attribute_prompt,
         token_prefix, token_suffix, tokenized_prompts,
         nc_token_prefix, nc_token_suffix, nc_tokenized_prompts):
    i = jnp.arange(B * CTX_PAD, dtype=jnp.int32).reshape(B, CTX_PAD)
    i = jnp.where(i % CTX_PAD < N_CTX,
                  (i // CTX_PAD) * N_CTX + i % CTX_PAD, 0)
    j = i % CTX_LEN
    idxg_rows = (idx_g[jnp.clip(i // CTX_LEN, 0, B - 1)] * CTX_LEN
                 + j).reshape(B, 1, CTX_PAD)
    ia = jnp.clip(i - G_ROWS, 0)
    idxa_rows = (idx_a[ia // CTX_LEN] * CTX_LEN
                 + ia % CTX_LEN).reshape(B, 1, CTX_PAD)

    sc_fn = pl.kernel(
        _sc_gather,
        out_type=jax.ShapeDtypeStruct((B, CTX_PAD, D), jnp.float32),
        mesh=plsc.VectorSubcoreMesh(core_axis_name="c", subcore_axis_name="s"),
        scratch_types=[
            pltpu.VMEM((CTX_PAD,), jnp.int32),
            pltpu.VMEM((CTX_PAD,), jnp.int32),
            pltpu.VMEM((CTX_PAD, D), jnp.float32),
            pltpu.SemaphoreType.DMA,
        ],
    )
    ctx_all = sc_fn(global_prompt.reshape(POOL_G * CTX_LEN, D),
                    attribute_prompt.reshape(POOL_A * CTX_LEN, D),
                    idxg_rows, idxa_rows)

    main_fn = pl.pallas_call(
        _prompt_kernel,
        grid=(NCB, B),
        in_specs=[
            pl.BlockSpec((B, CTX_PAD, D), lambda cb, b: (0, 0, 0)),
            pl.BlockSpec((CLS, 1, D), lambda cb, b: (0, 0, 0)),
            pl.BlockSpec((CLS, SUF, D), lambda cb, b: (0, 0, 0)),
            pl.BlockSpec((CLS, 1, SEQ), lambda cb, b: (0, 0, 0)),
        ],
        out_specs=[
            pl.BlockSpec((CLS_BLK, SEQ, D), lambda cb, b: (b * NCB + cb, 0, 0)),
            pl.BlockSpec((CLS_BLK, 1, SEQ), lambda cb, b: (b * NCB + cb, 0, 0)),
        ],
        out_shape=[
            jax.ShapeDtypeStruct((B * CLS, SEQ, D), jnp.float32),
            jax.ShapeDtypeStruct((B * CLS, 1, SEQ), jnp.int32),
        ],
        compiler_params=pltpu.CompilerParams(
            dimension_semantics=("parallel", "parallel")),
    )
    prompts, tok3 = main_fn(ctx_all, token_prefix, token_suffix,
                            tokenized_prompts.reshape(CLS, 1, SEQ))

    nc_fn = pl.pallas_call(
        _nc_kernel,
        out_shape=[
            jax.ShapeDtypeStruct((POOL_G, SEQ, D), jnp.float32),
            jax.ShapeDtypeStruct((POOL_G, SEQ), jnp.int32),
        ],
    )
    nc_prompts, nc_tok = nc_fn(global_prompt, nc_token_prefix,
                               nc_token_suffix, nc_tokenized_prompts)

    return prompts, tok3.reshape(B * CLS, SEQ), nc_prompts, nc_tok


def kernel(indices_g, indices_a, global_prompt, attribute_prompt,
           token_prefix, token_suffix, tokenized_prompts,
           nc_token_prefix, nc_token_suffix, nc_tokenized_prompts):
    idx_g = indices_g.astype(jnp.int32)
    idx_a = indices_a.astype(jnp.int32)
    return _run(idx_g, idx_a, global_prompt, attribute_prompt,
                token_prefix, token_suffix, tokenized_prompts,
                nc_token_prefix, nc_token_suffix, nc_tokenized_prompts)


# speedup vs baseline: 1.0368x; 1.0368x over previous
import jax
import jax.numpy as jnp
from jax import lax
from jax.experimental import pallas as pl
from jax.experimental.pallas import tpu as pltpu
from jax.experimental.pallas import tpu_sc as plsc

B = 32
CLS = 100
D = 512
CTX_LEN = 12
POOL_G = 10
POOL_A = 100
SEQ = 77
N_CTX = 36
SUF = 40
NC_SUF = 64

NCORE = 2
NSUB = 16
NW = NCORE * NSUB
G_ROWS = B * CTX_LEN
A_ROWS = 2 * B * CTX_LEN
GW = 24
GPT = G_ROWS // GW
APT = A_ROWS // NW

CLS_BLK = 50
NCB = CLS // CLS_BLK


def _sc_gather(g2d_hbm, a2d_hbm, idxg_hbm, idxa_hbm, ctx_hbm,
               idxg_v, idxa_v, gat_g, gat_a, sem):
    cid = lax.axis_index("c")
    sid = lax.axis_index("s")
    w = sid * NCORE + cid

    g_off = pl.multiple_of(GPT * w, 8)
    a_off = pl.multiple_of(APT * w, 8)
    a_dst = pl.multiple_of(G_ROWS + APT * w, 8)

    @pl.when(w < GW)
    def _gather_global():
        pltpu.sync_copy(idxg_hbm.at[pl.ds(g_off, GPT)], idxg_v)
        pltpu.async_copy(g2d_hbm.at[idxg_v], gat_g, sem).wait()
        pltpu.sync_copy(gat_g, ctx_hbm.at[pl.ds(g_off, GPT)])

    pltpu.sync_copy(idxa_hbm.at[pl.ds(a_off, APT)], idxa_v)
    pltpu.async_copy(a2d_hbm.at[idxa_v], gat_a, sem).wait()
    pltpu.sync_copy(gat_a, ctx_hbm.at[pl.ds(a_dst, APT)])


def _prompt_kernel(ctx_ref, pref_ref, suf_ref, tok_ref,
                   out_p_ref, out_t_ref):
    cb = pl.program_id(0)
    b = pl.program_id(1)
    c0 = cb * CLS_BLK

    ctx = ctx_ref[b]
    full = jnp.concatenate([
        pref_ref[pl.ds(c0, CLS_BLK)],
        jnp.broadcast_to(ctx[None], (CLS_BLK, N_CTX, D)),
        suf_ref[pl.ds(c0, CLS_BLK)],
    ], axis=1)
    out_p_ref[...] = full
    out_t_ref[...] = tok_ref[pl.ds(c0, CLS_BLK)]


def _nc_kernel(g_ref, ncp_ref, ncs_ref, nct_ref, out_ncp_ref, out_nct_ref):
    out_ncp_ref[...] = jnp.concatenate([
        jnp.broadcast_to(ncp_ref[...], (POOL_G, 1, D)),
        g_ref[...],
        jnp.broadcast_to(ncs_ref[...], (POOL_G, NC_SUF, D)),
    ], axis=1)
    out_nct_ref[...] = jnp.broadcast_to(nct_ref[...], (POOL_G, SEQ))


@jax.jit
def _run(idx_g, idx_a, global_prompt, attribute_prompt,
         token_prefix, token_suffix, tokenized_prompts,
         nc_token_prefix, nc_token_suffix, nc_tokenized_prompts):
    idxg_rows = (idx_g[:, None] * CTX_LEN
                 + jnp.arange(CTX_LEN, dtype=jnp.int32)[None, :]).reshape(G_ROWS)
    idxa_rows = (idx_a[:, None] * CTX_LEN
                 + jnp.arange(CTX_LEN, dtype=jnp.int32)[None, :]).reshape(A_ROWS)

    sc_fn = pl.kernel(
        _sc_gather,
        out_type=jax.ShapeDtypeStruct((G_ROWS + A_ROWS, D), jnp.float32),
        mesh=plsc.VectorSubcoreMesh(core_axis_name="c", subcore_axis_name="s"),
        scratch_types=[
            pltpu.VMEM((GPT,), jnp.int32),
            pltpu.VMEM((APT,), jnp.int32),
            pltpu.VMEM((GPT, D), jnp.float32),
            pltpu.VMEM((APT, D), jnp.float32),
            pltpu.SemaphoreType.DMA,
        ],
    )
    ctx_all = sc_fn(global_prompt.reshape(POOL_G * CTX_LEN, D),
                    attribute_prompt.reshape(POOL_A * CTX_LEN, D),
                    idxg_rows, idxa_rows)

    main_fn = pl.pallas_call(
        _prompt_kernel,
        grid=(NCB, B),
        in_specs=[
            pl.BlockSpec((B, N_CTX, D), lambda cb, b: (0, 0, 0)),
            pl.BlockSpec((CLS, 1, D), lambda cb, b: (0, 0, 0)),
            pl.BlockSpec((CLS, SUF, D), lambda cb, b: (0, 0, 0)),
            pl.BlockSpec((CLS, 1, SEQ), lambda cb, b: (0, 0, 0)),
        ],
        out_specs=[
            pl.BlockSpec((CLS_BLK, SEQ, D), lambda cb, b: (b * NCB + cb, 0, 0)),
            pl.BlockSpec((CLS_BLK, 1, SEQ), lambda cb, b: (b * NCB + cb, 0, 0)),
        ],
        out_shape=[
            jax.ShapeDtypeStruct((B * CLS, SEQ, D), jnp.float32),
            jax.ShapeDtypeStruct((B * CLS, 1, SEQ), jnp.int32),
        ],
        compiler_params=pltpu.CompilerParams(
            dimension_semantics=("parallel", "parallel")),
    )
    prompts, tok3 = main_fn(ctx_all.reshape(B, N_CTX, D),
                            token_prefix, token_suffix,
                            tokenized_prompts.reshape(CLS, 1, SEQ))

    nc_fn = pl.pallas_call(
        _nc_kernel,
        out_shape=[
            jax.ShapeDtypeStruct((POOL_G, SEQ, D), jnp.float32),
            jax.ShapeDtypeStruct((POOL_G, SEQ), jnp.int32),
        ],
    )
    nc_prompts, nc_tok = nc_fn(global_prompt, nc_token_prefix,
                               nc_token_suffix, nc_tokenized_prompts)

    return prompts, tok3.reshape(B * CLS, SEQ), nc_prompts, nc_tok


def kernel(indices_g, indices_a, global_prompt, attribute_prompt,
           token_prefix, token_suffix, tokenized_prompts,
           nc_token_prefix, nc_token_suffix, nc_tokenized_prompts):
    idx_g = indices_g.astype(jnp.int32)
    idx_a = indices_a.astype(jnp.int32)
    return _run(idx_g, idx_a, global_prompt, attribute_prompt,
                token_prefix, token_suffix, tokenized_prompts,
                nc_token_prefix, nc_token_suffix, nc_tokenized_prompts)
